# baseline (speedup 1.0000x reference)
import jax
import jax.numpy as jnp
from jax.experimental import pallas as pl

_THRESHOLD = 0.01


def _tc_body(x_ref, o_ref):
    x0 = x_ref[0]
    x1 = x_ref[1]
    n = x0.shape[-1]
    m0 = jnp.mean(x0, axis=1, keepdims=True)
    m1 = jnp.mean(x1, axis=1, keepdims=True)
    v0 = jnp.sum((x0 - m0) * (x0 - m0), axis=1, keepdims=True) / (n - 1)
    v1 = jnp.sum((x1 - m1) * (x1 - m1), axis=1, keepdims=True) / (n - 1)
    o_ref[0] = jnp.where(v0 >= _THRESHOLD, x0, x1)
    o_ref[1] = jnp.where(v1 >= _THRESHOLD, x1, x0)


def kernel(x):
    M, n, c, H, W = x.shape
    rows = n * c
    hw = H * W
    xf = x.reshape(M, rows, hw)
    R = 128
    out = pl.pallas_call(
        _tc_body,
        grid=(rows // R,),
        in_specs=[pl.BlockSpec((M, R, hw), lambda i: (0, i, 0))],
        out_specs=pl.BlockSpec((M, R, hw), lambda i: (0, i, 0)),
        out_shape=jax.ShapeDtypeStruct((M, rows, hw), jnp.float32),
    )(xf)
    return out.reshape(M, n, c, H, W)


# speedup vs baseline: 1.3288x; 1.3288x over previous
import jax
import jax.numpy as jnp
from jax.experimental import pallas as pl

_THRESHOLD = 0.01


def _tc_body(x_ref, o_ref):
    x0 = x_ref[0]
    x1 = x_ref[1]
    hw = x0.shape[-1] * x0.shape[-2]
    s0 = jnp.sum(x0, axis=(2, 3), keepdims=True)
    s1 = jnp.sum(x1, axis=(2, 3), keepdims=True)
    ss0 = jnp.sum(x0 * x0, axis=(2, 3), keepdims=True)
    ss1 = jnp.sum(x1 * x1, axis=(2, 3), keepdims=True)
    v0 = (ss0 - s0 * s0 * (1.0 / hw)) * (1.0 / (hw - 1))
    v1 = (ss1 - s1 * s1 * (1.0 / hw)) * (1.0 / (hw - 1))
    o_ref[0] = jnp.where(v0 >= _THRESHOLD, x0, x1)
    o_ref[1] = jnp.where(v1 >= _THRESHOLD, x1, x0)


def kernel(x):
    M, n, c, H, W = x.shape
    CB = 8
    out = pl.pallas_call(
        _tc_body,
        grid=(c // CB,),
        in_specs=[pl.BlockSpec((M, n, CB, H, W), lambda i: (0, 0, i, 0, 0))],
        out_specs=pl.BlockSpec((M, n, CB, H, W), lambda i: (0, 0, i, 0, 0)),
        out_shape=jax.ShapeDtypeStruct((M, n, c, H, W), jnp.float32),
    )(x)
    return out


# speedup vs baseline: 1.6906x; 1.2723x over previous
import jax
import jax.numpy as jnp
from jax.experimental import pallas as pl

_THRESHOLD = 0.01


def _tc_body(x_ref, o_ref):
    x0 = x_ref[0]
    x1 = x_ref[1]
    hw = x0.shape[-1] * x0.shape[-2]
    s0 = jnp.sum(x0, axis=(1, 2), keepdims=True)
    s1 = jnp.sum(x1, axis=(1, 2), keepdims=True)
    ss0 = jnp.sum(x0 * x0, axis=(1, 2), keepdims=True)
    ss1 = jnp.sum(x1 * x1, axis=(1, 2), keepdims=True)
    v0 = (ss0 - s0 * s0 * (1.0 / hw)) * (1.0 / (hw - 1))
    v1 = (ss1 - s1 * s1 * (1.0 / hw)) * (1.0 / (hw - 1))
    o_ref[0] = jnp.where(v0 >= _THRESHOLD, x0, x1)
    o_ref[1] = jnp.where(v1 >= _THRESHOLD, x1, x0)


def kernel(x):
    M, n, c, H, W = x.shape
    rows = n * c
    xf = x.reshape(M, rows, H, W)
    R = 256
    out = pl.pallas_call(
        _tc_body,
        grid=(rows // R,),
        in_specs=[pl.BlockSpec((M, R, H, W), lambda i: (0, i, 0, 0))],
        out_specs=pl.BlockSpec((M, R, H, W), lambda i: (0, i, 0, 0)),
        out_shape=jax.ShapeDtypeStruct((M, rows, H, W), jnp.float32),
    )(xf)
    return out.reshape(M, n, c, H, W)


# speedup vs baseline: 8.7047x; 5.1489x over previous
import jax
import jax.numpy as jnp
from jax.experimental import pallas as pl

_THRESHOLD = 0.01


def _tc_body(x_ref, o_ref):
    x0 = x_ref[0, 0]
    x1 = x_ref[1, 0]
    hw = x0.shape[0] * x0.shape[1]
    s0 = jnp.sum(x0, axis=(0, 1), keepdims=True)
    s1 = jnp.sum(x1, axis=(0, 1), keepdims=True)
    ss0 = jnp.sum(x0 * x0, axis=(0, 1), keepdims=True)
    ss1 = jnp.sum(x1 * x1, axis=(0, 1), keepdims=True)
    v0 = (ss0 - s0 * s0 * (1.0 / hw)) * (1.0 / (hw - 1))
    v1 = (ss1 - s1 * s1 * (1.0 / hw)) * (1.0 / (hw - 1))
    o_ref[0, 0] = jnp.where(v0 >= _THRESHOLD, x0, x1)
    o_ref[1, 0] = jnp.where(v1 >= _THRESHOLD, x1, x0)


def kernel(x):
    M, n, c, H, W = x.shape
    xt = jnp.transpose(x, (0, 1, 3, 4, 2))
    CB = 128
    out_t = pl.pallas_call(
        _tc_body,
        grid=(n, c // CB),
        in_specs=[pl.BlockSpec((M, 1, H, W, CB), lambda i, j: (0, i, 0, 0, j))],
        out_specs=pl.BlockSpec((M, 1, H, W, CB), lambda i, j: (0, i, 0, 0, j)),
        out_shape=jax.ShapeDtypeStruct((M, n, H, W, c), jnp.float32),
    )(xt)
    return jnp.transpose(out_t, (0, 1, 4, 2, 3))


# speedup vs baseline: 9.5894x; 1.1016x over previous
import jax
import jax.numpy as jnp
from jax.experimental import pallas as pl

_THRESHOLD = 0.01


def _tc_body(x_ref, o_ref):
    x0 = x_ref[0, 0]
    x1 = x_ref[1, 0]
    hw = x0.shape[0] * x0.shape[1]
    s0 = jnp.sum(x0, axis=(0, 1), keepdims=True)
    s1 = jnp.sum(x1, axis=(0, 1), keepdims=True)
    ss0 = jnp.sum(x0 * x0, axis=(0, 1), keepdims=True)
    ss1 = jnp.sum(x1 * x1, axis=(0, 1), keepdims=True)
    v0 = (ss0 - s0 * s0 * (1.0 / hw)) * (1.0 / (hw - 1))
    v1 = (ss1 - s1 * s1 * (1.0 / hw)) * (1.0 / (hw - 1))
    o_ref[0, 0] = jnp.where(v0 >= _THRESHOLD, x0, x1)
    o_ref[1, 0] = jnp.where(v1 >= _THRESHOLD, x1, x0)


def kernel(x):
    M, n, c, H, W = x.shape
    xt = jnp.transpose(x, (0, 1, 3, 4, 2))
    CB = 384
    out_t = pl.pallas_call(
        _tc_body,
        grid=(n, c // CB),
        in_specs=[pl.BlockSpec((M, 1, H, W, CB), lambda i, j: (0, i, 0, 0, j))],
        out_specs=pl.BlockSpec((M, 1, H, W, CB), lambda i, j: (0, i, 0, 0, j)),
        out_shape=jax.ShapeDtypeStruct((M, n, H, W, c), jnp.float32),
    )(xt)
    return jnp.transpose(out_t, (0, 1, 4, 2, 3))
